# baseline (speedup 1.0000x reference)
import functools

import jax
import jax.numpy as jnp
from jax import lax
from jax.experimental import pallas as pl
from jax.experimental.pallas import tpu as pltpu
from jax.experimental.pallas import tpu_sc as plsc

N_NODES = 10000
HIDDEN = 128
NC, NS = 2, 16
NW = NC * NS
CHUNK = 128
N_PAD = 10112
ROWS_PER_TILE = N_PAD // NS


@functools.lru_cache(maxsize=None)
def _make_sc_kernel(ept, n_chunks):
  mesh = plsc.VectorSubcoreMesh(core_axis_name="c", subcore_axis_name="s")

  @functools.partial(
      pl.kernel,
      mesh=mesh,
      compiler_params=pltpu.CompilerParams(needs_layout_passes=False),
      out_type=(
          jax.ShapeDtypeStruct((NC, N_PAD, HIDDEN), jnp.float32),
          jax.ShapeDtypeStruct((NW * N_PAD,), jnp.float32),
      ),
      scratch_types=[
          pltpu.VMEM((2, CHUNK), jnp.int32),
          pltpu.VMEM((2, CHUNK), jnp.int32),
          pltpu.VMEM((2, CHUNK, HIDDEN), jnp.float32),
          pltpu.VMEM((N_PAD,), jnp.float32),
          pltpu.VMEM_SHARED((N_PAD, HIDDEN), jnp.float32),
          pltpu.SemaphoreType.DMA,
          pltpu.SemaphoreType.DMA,
          pltpu.SemaphoreType.DMA,
          pltpu.SemaphoreType.DMA,
      ],
  )
  def sc_agg(x_hbm, zeros_hbm, row_hbm, col_hbm, g_out, cnt_out,
             ridx, cidx, rows, cnt_loc, acc, gsem0, gsem1, ssem0, ssem1):
    cid = lax.axis_index("c")
    sid = lax.axis_index("s")
    wid = cid * NS + sid
    r0 = sid * ROWS_PER_TILE

    pltpu.sync_copy(zeros_hbm.at[pl.ds(r0, ROWS_PER_TILE)],
                    acc.at[pl.ds(r0, ROWS_PER_TILE)])
    zero16 = jnp.zeros((16,), jnp.float32)

    def _zero_cnt(i, carry):
      cnt_loc[pl.ds(i * 16, 16)] = zero16
      return carry

    lax.fori_loop(0, N_PAD // 16, _zero_cnt, 0)
    plsc.subcore_barrier()

    base = wid * ept
    ones16 = jnp.ones((16,), jnp.float32)

    def _edge_chunk(i, carry):
      off = base + i * CHUNK
      pltpu.sync_copy(row_hbm.at[pl.ds(off, CHUNK)], ridx.at[0])
      pltpu.sync_copy(col_hbm.at[pl.ds(off, CHUNK)], cidx.at[0])
      pltpu.async_copy(x_hbm.at[ridx.at[0]], rows.at[0], gsem0).wait()
      pltpu.sync_copy(rows.at[0], acc.at[cidx.at[0]], add=True)

      def _cnt(j, c2):
        idx16 = cidx[0, pl.ds(j * 16, 16)]
        plsc.addupdate_scatter(cnt_loc, [idx16], ones16)
        return c2

      lax.fori_loop(0, CHUNK // 16, _cnt, 0)
      return carry

    lax.fori_loop(0, n_chunks, _edge_chunk, 0)
    plsc.subcore_barrier()

    pltpu.sync_copy(acc.at[pl.ds(r0, ROWS_PER_TILE)],
                    g_out.at[cid, pl.ds(r0, ROWS_PER_TILE)])
    pltpu.sync_copy(cnt_loc, cnt_out.at[pl.ds(wid * N_PAD, N_PAD)])

  return sc_agg


def _tc_body(x_ref, g_ref, cnt_ref, w1_ref, b1_ref, w2_ref, b2_ref, s_ref,
             out_ref):
  x = x_ref[...]
  g = g_ref[0] + g_ref[1]
  cnt = jnp.sum(cnt_ref[...], axis=0)
  w1 = w1_ref[...]
  w2 = w2_ref[...]
  b1 = b1_ref[...]
  b2 = b2_ref[...]
  dn = (((1,), (1,)), ((), ()))
  xt = lax.dot_general(x, w1, dn, preferred_element_type=jnp.float32) + b1
  w21 = jnp.dot(w2, w1, preferred_element_type=jnp.float32)
  s = lax.dot_general(g, w21, dn, preferred_element_type=jnp.float32)
  d = lax.dot_general(b1, w2, dn, preferred_element_type=jnp.float32) + b2
  denom = jnp.maximum(cnt, 1.0)[:, None]
  mean = (s + cnt[:, None] * d) / denom
  sig = 1.0 / (1.0 + jnp.exp(-s_ref[0, 0]))
  out_ref[...] = xt - sig * mean


def kernel(x, edge_index, W1, b1, W2, b2, anti_strength):
  n_edges = edge_index.shape[1]
  ept_raw = -(-n_edges // NW)
  n_chunks = 2 * -(-ept_raw // (2 * CHUNK))
  ept = n_chunks * CHUNK
  e_pad = ept * NW

  row = edge_index[0].astype(jnp.int32)
  col = edge_index[1].astype(jnp.int32)
  dummy = N_NODES + jnp.arange(e_pad, dtype=jnp.int32) % (N_PAD - N_NODES)
  row_pad = jnp.zeros((e_pad,), jnp.int32).at[:n_edges].set(row)
  col_pad = dummy.at[:n_edges].set(col)
  x_pad = jnp.zeros((N_PAD, HIDDEN), jnp.float32).at[:N_NODES].set(x)
  zeros_pad = jnp.zeros((N_PAD, HIDDEN), jnp.float32)

  g_partial, cnt_partial = _make_sc_kernel(ept, n_chunks)(
      x_pad, zeros_pad, row_pad, col_pad)
  cnt_partial = cnt_partial.reshape(NW, N_PAD)

  out = pl.pallas_call(
      _tc_body,
      out_shape=jax.ShapeDtypeStruct((N_PAD, HIDDEN), jnp.float32),
  )(x_pad, g_partial, cnt_partial, W1, b1.reshape(1, HIDDEN), W2,
    b2.reshape(1, HIDDEN), anti_strength.reshape(1, 1))

  return out[:N_NODES]


# speedup vs baseline: 2.1723x; 2.1723x over previous
import functools

import jax
import jax.numpy as jnp
from jax import lax
from jax.experimental import pallas as pl
from jax.experimental.pallas import tpu as pltpu
from jax.experimental.pallas import tpu_sc as plsc

N_NODES = 10000
HIDDEN = 128
NC, NS = 2, 16
NW = NC * NS
CHUNK = 128
N_PAD = 10112
ROWS_PER_TILE = N_PAD // NS


@functools.lru_cache(maxsize=None)
def _make_sc_kernel(base_chunks, rem_chunks):
  mesh = plsc.VectorSubcoreMesh(core_axis_name="c", subcore_axis_name="s")

  @functools.partial(
      pl.kernel,
      mesh=mesh,
      compiler_params=pltpu.CompilerParams(needs_layout_passes=False),
      out_type=(
          jax.ShapeDtypeStruct((NC, N_PAD, HIDDEN), jnp.float32),
          jax.ShapeDtypeStruct((NW * N_PAD,), jnp.float32),
      ),
      scratch_types=[
          pltpu.VMEM((CHUNK,), jnp.int32),
          pltpu.VMEM((CHUNK,), jnp.int32),
          pltpu.VMEM((CHUNK, HIDDEN), jnp.float32),
          pltpu.VMEM((N_PAD,), jnp.float32),
          pltpu.VMEM_SHARED((N_PAD, HIDDEN), jnp.float32),
          pltpu.SemaphoreType.DMA,
      ],
  )
  def sc_agg(x_hbm, zeros_hbm, row_hbm, col_hbm, g_out, cnt_out,
             ridx, cidx, rows, cnt_loc, acc, gsem):
    cid = lax.axis_index("c")
    sid = lax.axis_index("s")
    wid = cid * NS + sid
    r0 = sid * ROWS_PER_TILE

    pltpu.sync_copy(zeros_hbm.at[pl.ds(r0, ROWS_PER_TILE)],
                    acc.at[pl.ds(r0, ROWS_PER_TILE)])
    zero16 = jnp.zeros((16,), jnp.float32)

    def _zero_cnt(i, carry):
      cnt_loc[pl.ds(i * 16, 16)] = zero16
      return carry

    lax.fori_loop(0, N_PAD // 16, _zero_cnt, 0)
    plsc.subcore_barrier()

    n_chunks = base_chunks + jnp.where(wid < rem_chunks, 1, 0)
    base = (wid * base_chunks + jnp.minimum(wid, rem_chunks)) * CHUNK
    ones16 = jnp.ones((16,), jnp.float32)

    def _edge_chunk(i, carry):
      off = base + i * CHUNK
      pltpu.sync_copy(row_hbm.at[pl.ds(off, CHUNK)], ridx)
      pltpu.sync_copy(col_hbm.at[pl.ds(off, CHUNK)], cidx)
      pltpu.async_copy(x_hbm.at[ridx], rows, gsem).wait()
      pltpu.sync_copy(rows, acc.at[cidx], add=True)

      def _cnt(j, c2):
        idx16 = cidx[pl.ds(j * 16, 16)]
        plsc.addupdate_scatter(cnt_loc, [idx16], ones16)
        return c2

      lax.fori_loop(0, CHUNK // 16, _cnt, 0)
      return carry

    lax.fori_loop(0, n_chunks, _edge_chunk, 0)
    plsc.subcore_barrier()

    pltpu.sync_copy(acc.at[pl.ds(r0, ROWS_PER_TILE)],
                    g_out.at[cid, pl.ds(r0, ROWS_PER_TILE)])
    pltpu.sync_copy(cnt_loc, cnt_out.at[pl.ds(wid * N_PAD, N_PAD)])

  return sc_agg


def _tc_body(x_ref, g_ref, cnt_ref, w1_ref, b1_ref, w2_ref, b2_ref, s_ref,
             out_ref):
  x = x_ref[...]
  g = g_ref[0] + g_ref[1]
  cnt = jnp.sum(cnt_ref[...], axis=0)
  w1 = w1_ref[...]
  w2 = w2_ref[...]
  b1 = b1_ref[...]
  b2 = b2_ref[...]
  dn = (((1,), (1,)), ((), ()))
  xt = lax.dot_general(x, w1, dn, preferred_element_type=jnp.float32) + b1
  w21 = jnp.dot(w2, w1, preferred_element_type=jnp.float32)
  s = lax.dot_general(g, w21, dn, preferred_element_type=jnp.float32)
  d = lax.dot_general(b1, w2, dn, preferred_element_type=jnp.float32) + b2
  denom = jnp.maximum(cnt, 1.0)[:, None]
  mean = (s + cnt[:, None] * d) / denom
  sig = 1.0 / (1.0 + jnp.exp(-s_ref[0, 0]))
  out_ref[...] = xt - sig * mean


def kernel(x, edge_index, W1, b1, W2, b2, anti_strength):
  n_edges = edge_index.shape[1]
  total_chunks = -(-n_edges // CHUNK)
  e_pad = total_chunks * CHUNK
  base_chunks, rem_chunks = divmod(total_chunks, NW)

  row = edge_index[0].astype(jnp.int32)
  col = edge_index[1].astype(jnp.int32)
  if e_pad > n_edges:
    dummy = N_NODES + jnp.arange(e_pad, dtype=jnp.int32) % (N_PAD - N_NODES)
    row_pad = dummy.at[:n_edges].set(row)
    col_pad = dummy.at[:n_edges].set(col)
  else:
    row_pad, col_pad = row, col
  x_pad = jnp.zeros((N_PAD, HIDDEN), jnp.float32).at[:N_NODES].set(x)
  zeros_pad = jnp.zeros((N_PAD, HIDDEN), jnp.float32)

  g_partial, cnt_partial = _make_sc_kernel(base_chunks, rem_chunks)(
      x_pad, zeros_pad, row_pad, col_pad)
  cnt_partial = cnt_partial.reshape(NW, N_PAD)

  out = pl.pallas_call(
      _tc_body,
      out_shape=jax.ShapeDtypeStruct((N_PAD, HIDDEN), jnp.float32),
  )(x_pad, g_partial, cnt_partial, W1, b1.reshape(1, HIDDEN), W2,
    b2.reshape(1, HIDDEN), anti_strength.reshape(1, 1))

  return out[:N_NODES]


# speedup vs baseline: 2.2656x; 1.0429x over previous
import functools

import jax
import jax.numpy as jnp
from jax import lax
from jax.experimental import pallas as pl
from jax.experimental.pallas import tpu as pltpu
from jax.experimental.pallas import tpu_sc as plsc

N_NODES = 10000
HIDDEN = 128
NC, NS = 2, 16
NW = NC * NS
CHUNK = 128
N_PAD = 10112
ROWS_PER_TILE = N_PAD // NS


@functools.lru_cache(maxsize=None)
def _make_sc_kernel(base_chunks, rem_chunks):
  mesh = plsc.VectorSubcoreMesh(core_axis_name="c", subcore_axis_name="s")

  @functools.partial(
      pl.kernel,
      mesh=mesh,
      compiler_params=pltpu.CompilerParams(needs_layout_passes=False),
      out_type=(
          jax.ShapeDtypeStruct((NC, N_PAD, HIDDEN), jnp.float32),
          jax.ShapeDtypeStruct((NW * N_PAD,), jnp.float32),
      ),
      scratch_types=[
          pltpu.VMEM((CHUNK,), jnp.int32),
          pltpu.VMEM((CHUNK,), jnp.int32),
          pltpu.VMEM((CHUNK, HIDDEN), jnp.float32),
          pltpu.VMEM((N_PAD,), jnp.float32),
          pltpu.VMEM_SHARED((N_PAD, HIDDEN), jnp.float32),
          pltpu.SemaphoreType.DMA,
      ],
  )
  def sc_agg(x_hbm, row_hbm, col_hbm, g_out, cnt_out,
             ridx, cidx, rows, cnt_loc, acc, gsem):
    cid = lax.axis_index("c")
    sid = lax.axis_index("s")
    wid = cid * NS + sid
    r0 = sid * ROWS_PER_TILE

    zero16 = jnp.zeros((16,), jnp.float32)

    def _zero_rows(i, carry):
      def _zr(j, c2):
        rows[i, pl.ds(j * 16, 16)] = zero16
        return c2

      lax.fori_loop(0, HIDDEN // 16, _zr, 0)
      return carry

    lax.fori_loop(0, CHUNK, _zero_rows, 0)

    def _zero_cnt(i, carry):
      cnt_loc[pl.ds(i * 16, 16)] = zero16
      return carry

    lax.fori_loop(0, N_PAD // 16, _zero_cnt, 0)

    n_full, n_tail = divmod(ROWS_PER_TILE, CHUNK)
    for kk in range(n_full):
      pltpu.sync_copy(rows, acc.at[pl.ds(r0 + kk * CHUNK, CHUNK)])
    if n_tail:
      pltpu.sync_copy(rows.at[pl.ds(0, n_tail)],
                      acc.at[pl.ds(r0 + n_full * CHUNK, n_tail)])
    plsc.subcore_barrier()

    n_chunks = base_chunks + jnp.where(wid < rem_chunks, 1, 0)
    base = (wid * base_chunks + jnp.minimum(wid, rem_chunks)) * CHUNK
    ones16 = jnp.ones((16,), jnp.float32)

    def _edge_chunk(i, carry):
      off = base + i * CHUNK
      pltpu.sync_copy(row_hbm.at[pl.ds(off, CHUNK)], ridx)
      pltpu.sync_copy(col_hbm.at[pl.ds(off, CHUNK)], cidx)
      pltpu.async_copy(x_hbm.at[ridx], rows, gsem).wait()
      pltpu.sync_copy(rows, acc.at[cidx], add=True)

      def _cnt(j, c2):
        idx16 = cidx[pl.ds(j * 16, 16)]
        plsc.addupdate_scatter(cnt_loc, [idx16], ones16)
        return c2

      lax.fori_loop(0, CHUNK // 16, _cnt, 0)
      return carry

    lax.fori_loop(0, n_chunks, _edge_chunk, 0)
    plsc.subcore_barrier()

    pltpu.sync_copy(acc.at[pl.ds(r0, ROWS_PER_TILE)],
                    g_out.at[cid, pl.ds(r0, ROWS_PER_TILE)])
    pltpu.sync_copy(cnt_loc, cnt_out.at[pl.ds(wid * N_PAD, N_PAD)])

  return sc_agg


def _tc_body(x_ref, g_ref, cnt_ref, w1_ref, b1_ref, w2_ref, b2_ref, s_ref,
             out_ref):
  x = x_ref[...]
  g = g_ref[0, :N_NODES, :] + g_ref[1, :N_NODES, :]
  cnt = jnp.sum(cnt_ref[...], axis=0)[:N_NODES]
  w1 = w1_ref[...]
  w2 = w2_ref[...]
  b1 = b1_ref[...]
  b2 = b2_ref[...]
  dn = (((1,), (1,)), ((), ()))
  xt = lax.dot_general(x, w1, dn, preferred_element_type=jnp.float32) + b1
  w21 = jnp.dot(w2, w1, preferred_element_type=jnp.float32)
  s = lax.dot_general(g, w21, dn, preferred_element_type=jnp.float32)
  d = lax.dot_general(b1, w2, dn, preferred_element_type=jnp.float32) + b2
  denom = jnp.maximum(cnt, 1.0)[:, None]
  mean = (s + cnt[:, None] * d) / denom
  sig = 1.0 / (1.0 + jnp.exp(-s_ref[0, 0]))
  out_ref[...] = xt - sig * mean


def kernel(x, edge_index, W1, b1, W2, b2, anti_strength):
  n_edges = edge_index.shape[1]
  total_chunks = -(-n_edges // CHUNK)
  e_pad = total_chunks * CHUNK
  base_chunks, rem_chunks = divmod(total_chunks, NW)

  row = edge_index[0].astype(jnp.int32)
  col = edge_index[1].astype(jnp.int32)
  if e_pad > n_edges:
    dummy = N_NODES + jnp.arange(e_pad, dtype=jnp.int32) % (N_PAD - N_NODES)
    row_pad = dummy.at[:n_edges].set(row)
    col_pad = dummy.at[:n_edges].set(col)
  else:
    row_pad, col_pad = row, col
  if e_pad > n_edges:
    x_table = jnp.zeros((N_PAD, HIDDEN), jnp.float32).at[:N_NODES].set(x)
  else:
    x_table = x

  g_partial, cnt_partial = _make_sc_kernel(base_chunks, rem_chunks)(
      x_table, row_pad, col_pad)
  cnt_partial = cnt_partial.reshape(NW, N_PAD)

  return pl.pallas_call(
      _tc_body,
      out_shape=jax.ShapeDtypeStruct((N_NODES, HIDDEN), jnp.float32),
  )(x, g_partial, cnt_partial, W1, b1.reshape(1, HIDDEN), W2,
    b2.reshape(1, HIDDEN), anti_strength.reshape(1, 1))


# speedup vs baseline: 3.3095x; 1.4608x over previous
import functools

import jax
import jax.numpy as jnp
from jax import lax
from jax.experimental import pallas as pl
from jax.experimental.pallas import tpu as pltpu
from jax.experimental.pallas import tpu_sc as plsc

N_NODES = 10000
HIDDEN = 128
NC, NS = 2, 16
NW = NC * NS
CHUNK = 128
N_PAD = 10112
ROWS_PER_TILE = N_PAD // NS


DEPTH = 2


@functools.lru_cache(maxsize=None)
def _make_sc_kernel(base_quads, quad_rem_tiles, tail_chunks):
  mesh = plsc.VectorSubcoreMesh(core_axis_name="c", subcore_axis_name="s")

  @functools.partial(
      pl.kernel,
      mesh=mesh,
      compiler_params=pltpu.CompilerParams(needs_layout_passes=False),
      out_type=(
          jax.ShapeDtypeStruct((NC, N_PAD, HIDDEN), jnp.float32),
          jax.ShapeDtypeStruct((NW * N_PAD,), jnp.float32),
      ),
      scratch_types=[
          pltpu.VMEM((DEPTH, CHUNK), jnp.int32),
          pltpu.VMEM((DEPTH, CHUNK), jnp.int32),
          pltpu.VMEM((DEPTH, CHUNK, HIDDEN), jnp.float32),
          pltpu.VMEM((N_PAD,), jnp.float32),
          pltpu.VMEM_SHARED((N_PAD, HIDDEN), jnp.float32),
          [pltpu.SemaphoreType.DMA] * DEPTH,
          [pltpu.SemaphoreType.DMA] * DEPTH,
      ],
  )
  def sc_agg(x_hbm, row_hbm, col_hbm, g_out, cnt_out,
             ridx, cidx, rows, cnt_loc, acc, gsems, ssems):
    cid = lax.axis_index("c")
    sid = lax.axis_index("s")
    wid = cid * NS + sid
    r0 = sid * ROWS_PER_TILE

    zero16 = jnp.zeros((16,), jnp.float32)

    def _zero_rows(i, carry):
      def _zr(j, c2):
        rows[0, i, pl.ds(j * 16, 16)] = zero16
        return c2

      lax.fori_loop(0, HIDDEN // 16, _zr, 0)
      return carry

    lax.fori_loop(0, CHUNK, _zero_rows, 0)

    def _zero_cnt(i, carry):
      cnt_loc[pl.ds(i * 16, 16)] = zero16
      return carry

    lax.fori_loop(0, N_PAD // 16, _zero_cnt, 0)

    n_full, n_tail = divmod(ROWS_PER_TILE, CHUNK)
    for kk in range(n_full):
      pltpu.sync_copy(rows.at[0], acc.at[pl.ds(r0 + kk * CHUNK, CHUNK)])
    if n_tail:
      pltpu.sync_copy(rows.at[0, pl.ds(0, n_tail)],
                      acc.at[pl.ds(r0 + n_full * CHUNK, n_tail)])
    plsc.subcore_barrier()

    n_quads = base_quads + jnp.where(wid < quad_rem_tiles, 1, 0)
    start_chunk = (base_quads * DEPTH * wid
                   + DEPTH * jnp.minimum(wid, quad_rem_tiles)
                   + tail_chunks * jnp.where(wid > quad_rem_tiles, 1, 0))
    base = start_chunk * CHUNK
    ones16 = jnp.ones((16,), jnp.float32)

    def _load_and_fire(b, chunk):
      off = base + chunk * CHUNK
      pltpu.sync_copy(row_hbm.at[pl.ds(off, CHUNK)], ridx.at[b])
      pltpu.sync_copy(col_hbm.at[pl.ds(off, CHUNK)], cidx.at[b])
      pltpu.async_copy(x_hbm.at[ridx.at[b]], rows.at[b], gsems[b])

    def _counts(b):
      def _cnt(j, c2):
        idx16 = cidx[b, pl.ds(j * 16, 16)]
        plsc.addupdate_scatter(cnt_loc, [idx16], ones16)
        return c2

      lax.fori_loop(0, CHUNK // 16, _cnt, 0)

    for b in range(DEPTH):
      _load_and_fire(b, b)

    def _quad(q, carry):
      for b in range(DEPTH):
        pltpu.make_async_copy(x_hbm.at[ridx.at[b]], rows.at[b],
                              gsems[b]).wait()
        pltpu.async_copy(rows.at[b], acc.at[cidx.at[b]], ssems[b], add=True)
        _counts(b)
      for b in range(DEPTH):
        pltpu.make_async_copy(rows.at[b], acc.at[cidx.at[b]],
                              ssems[b]).wait()
        _load_and_fire(b, (q + 1) * DEPTH + b)
      return carry

    lax.fori_loop(0, n_quads, _quad, 0)

    for b in range(DEPTH):
      pltpu.make_async_copy(x_hbm.at[ridx.at[b]], rows.at[b],
                            gsems[b]).wait()
    if tail_chunks:

      @pl.when(wid == quad_rem_tiles)
      def _tail():
        for b in range(tail_chunks):
          pltpu.sync_copy(rows.at[b], acc.at[cidx.at[b]], add=True)
          _counts(b)

    plsc.subcore_barrier()

    pltpu.sync_copy(acc.at[pl.ds(r0, ROWS_PER_TILE)],
                    g_out.at[cid, pl.ds(r0, ROWS_PER_TILE)])
    pltpu.sync_copy(cnt_loc, cnt_out.at[pl.ds(wid * N_PAD, N_PAD)])

  return sc_agg


def _tc_body(x_ref, g_ref, cnt_ref, w1_ref, b1_ref, w2_ref, b2_ref, s_ref,
             out_ref):
  x = x_ref[...]
  g = g_ref[0, :N_NODES, :] + g_ref[1, :N_NODES, :]
  cnt = jnp.sum(cnt_ref[...], axis=0)[:N_NODES]
  w1 = w1_ref[...]
  w2 = w2_ref[...]
  b1 = b1_ref[...]
  b2 = b2_ref[...]
  dn = (((1,), (1,)), ((), ()))
  xt = lax.dot_general(x, w1, dn, preferred_element_type=jnp.float32) + b1
  w21 = jnp.dot(w2, w1, preferred_element_type=jnp.float32)
  s = lax.dot_general(g, w21, dn, preferred_element_type=jnp.float32)
  d = lax.dot_general(b1, w2, dn, preferred_element_type=jnp.float32) + b2
  denom = jnp.maximum(cnt, 1.0)[:, None]
  mean = (s + cnt[:, None] * d) / denom
  sig = 1.0 / (1.0 + jnp.exp(-s_ref[0, 0]))
  out_ref[...] = xt - sig * mean


def kernel(x, edge_index, W1, b1, W2, b2, anti_strength):
  n_edges = edge_index.shape[1]
  total_chunks = -(-n_edges // CHUNK)
  e_pad = total_chunks * CHUNK
  per_tile = total_chunks // NW
  base_quads, _ = divmod(per_tile, DEPTH)
  rem = total_chunks - base_quads * DEPTH * NW
  quad_rem_tiles, tail_chunks = divmod(rem, DEPTH)

  row = edge_index[0].astype(jnp.int32)
  col = edge_index[1].astype(jnp.int32)
  if e_pad > n_edges:
    dummy = N_NODES + jnp.arange(e_pad, dtype=jnp.int32) % (N_PAD - N_NODES)
    row_pad = dummy.at[:n_edges].set(row)
    col_pad = dummy.at[:n_edges].set(col)
    x_table = jnp.zeros((N_PAD, HIDDEN), jnp.float32).at[:N_NODES].set(x)
  else:
    row_pad, col_pad = row, col
    x_table = x
  overrun = jnp.arange(DEPTH * CHUNK, dtype=jnp.int32) % N_NODES
  row_pad = jnp.concatenate([row_pad, overrun])
  col_pad = jnp.concatenate([col_pad, overrun])

  g_partial, cnt_partial = _make_sc_kernel(
      base_quads, quad_rem_tiles, tail_chunks)(x_table, row_pad, col_pad)
  cnt_partial = cnt_partial.reshape(NW, N_PAD)

  return pl.pallas_call(
      _tc_body,
      out_shape=jax.ShapeDtypeStruct((N_NODES, HIDDEN), jnp.float32),
  )(x, g_partial, cnt_partial, W1, b1.reshape(1, HIDDEN), W2,
    b2.reshape(1, HIDDEN), anti_strength.reshape(1, 1))
